# Initial kernel scaffold; baseline (speedup 1.0000x reference)
import jax
import jax.numpy as jnp
from jax.experimental import pallas as pl


def kernel(x, coords, edge_index, edge_attr, params):
    raise NotImplementedError("write your pallas kernel here")


# speedup vs baseline: 2.5881x; 2.5881x over previous
import functools

import jax
import jax.numpy as jnp
from jax import lax
from jax.experimental import pallas as pl
from jax.experimental.pallas import tpu as pltpu
from jax.experimental.pallas import tpu_sc as plsc

N = 10000
E = 320000
H = 128
NUM_LAYERS = 3

NC, NS = 2, 16
NW = NC * NS
CH = 80
EPT = E // NW
NCH = EPT // CH
NPAD = 10240
NPT = NPAD // NS
L = 16

BLKE = 512
BLKN = 512


@functools.cache
def _mesh():
    return plsc.VectorSubcoreMesh(
        core_axis_name="c", subcore_axis_name="s", num_cores=NC, num_subcores=NS)


def _silu(v):
    return v * jax.nn.sigmoid(v)


def _sc_gather_body(ta_hbm, tb_hbm, cflat_hbm, dst_hbm, src_hbm,
                    otd_hbm, ots_hbm, ord_hbm,
                    idx_d, idx_s, td, ts, cds, rd, sem_d, sem_s):
    cid = lax.axis_index("c")
    sid = lax.axis_index("s")
    wid = cid * NS + sid
    pltpu.sync_copy(dst_hbm.at[wid], idx_d)
    pltpu.sync_copy(src_hbm.at[wid], idx_s)
    pltpu.sync_copy(cflat_hbm, cds)
    ebase = wid * EPT
    lanes = lax.iota(jnp.int32, L)

    def body(c, carry):
        cpy_d = pltpu.async_copy(ta_hbm.at[idx_d.at[c]], td, sem_d)
        cpy_s = pltpu.async_copy(tb_hbm.at[idx_s.at[c]], ts, sem_s)

        def grp(g, carry2):
            dv = idx_d[c, pl.ds(g * L, L)] * 4
            sv = idx_s[c, pl.ds(g * L, L)] * 4
            erow = lanes + g * L
            dist = jnp.zeros((L,), jnp.float32)
            for comp in range(3):
                pd = plsc.load_gather(cds, [dv + comp])
                ps = plsc.load_gather(cds, [sv + comp])
                rel = pd - ps
                dist = dist + rel * rel
                plsc.store_scatter(rd, [erow, jnp.full((L,), comp, jnp.int32)], rel)
            plsc.store_scatter(rd, [erow, jnp.full((L,), 3, jnp.int32)], dist)
            return carry2

        lax.fori_loop(0, CH // L, grp, 0)
        pltpu.sync_copy(rd, ord_hbm.at[pl.ds(ebase + c * CH, CH)])
        cpy_d.wait()
        pltpu.sync_copy(td, otd_hbm.at[pl.ds(ebase + c * CH, CH)])
        cpy_s.wait()
        pltpu.sync_copy(ts, ots_hbm.at[pl.ds(ebase + c * CH, CH)])
        return carry

    lax.fori_loop(0, NCH, body, 0)


@functools.cache
def _sc_gather_kernel():
    return pl.kernel(
        _sc_gather_body,
        out_type=[
            jax.ShapeDtypeStruct((E, H), jnp.float32),
            jax.ShapeDtypeStruct((E, H), jnp.float32),
            jax.ShapeDtypeStruct((E, 16), jnp.float32),
        ],
        mesh=_mesh(),
        scratch_types=[
            pltpu.VMEM((NCH, CH), jnp.int32),
            pltpu.VMEM((NCH, CH), jnp.int32),
            pltpu.VMEM((CH, H), jnp.float32),
            pltpu.VMEM((CH, H), jnp.float32),
            pltpu.VMEM((4 * N,), jnp.float32),
            pltpu.VMEM((CH, 16), jnp.float32),
            pltpu.SemaphoreType.DMA,
            pltpu.SemaphoreType.DMA,
        ],
        compiler_params=pltpu.CompilerParams(needs_layout_passes=False),
    )


def _sc_gather(ta, tb, cflat, dst3d, src3d):
    return _sc_gather_kernel()(ta, tb, cflat, dst3d, src3d)


HH = H // 2
ECH = E // NS // CH
IDXB = 50
NIB = ECH // IDXB


def _sc_scatter_body(p2_hbm, aux_hbm, dst_hbm, zh_hbm, zs_hbm,
                     ulo_hbm, uhi_hbm, sd_hbm,
                     idx_d, idxrow, pbuf, abuf, uh, sdh):
    cid = lax.axis_index("c")
    sid = lax.axis_index("s")

    @pl.when(sid == 0)
    def _():
        pltpu.sync_copy(zh_hbm, uh)
        pltpu.sync_copy(zs_hbm, sdh)

    plsc.subcore_barrier()

    ebase = sid * (E // NS)
    pbase = cid * E + ebase
    pltpu.sync_copy(dst_hbm.at[sid], idx_d)

    def body(c, carry):
        rows = pl.ds(pbase + c * CH, CH)
        arows = pl.ds(ebase + c * CH, CH)
        for g in range(CH // L):
            idxrow[pl.ds(g * L, L)] = idx_d[c, pl.ds(g * L, L)]
        pltpu.sync_copy(p2_hbm.at[rows], pbuf)
        pltpu.sync_copy(pbuf, uh.at[idxrow], add=True)
        pltpu.sync_copy(aux_hbm.at[arows], abuf)
        pltpu.sync_copy(abuf, sdh.at[idxrow], add=True)
        return carry

    lax.fori_loop(0, ECH, body, 0)
    plsc.subcore_barrier()

    @pl.when((sid == 0) & (cid == 0))
    def _():
        pltpu.sync_copy(uh, ulo_hbm)

    @pl.when((sid == 0) & (cid == 1))
    def _():
        pltpu.sync_copy(uh, uhi_hbm)
        pltpu.sync_copy(sdh, sd_hbm)


@functools.cache
def _sc_scatter_kernel():
    return pl.kernel(
        _sc_scatter_body,
        out_type=[
            jax.ShapeDtypeStruct((NPAD, HH), jnp.float32),
            jax.ShapeDtypeStruct((NPAD, HH), jnp.float32),
            jax.ShapeDtypeStruct((NPAD, 16), jnp.float32),
        ],
        mesh=_mesh(),
        scratch_types=[
            pltpu.VMEM((ECH, CH), jnp.int32),
            pltpu.VMEM((CH,), jnp.int32),
            pltpu.VMEM((CH, HH), jnp.float32),
            pltpu.VMEM((CH, 16), jnp.float32),
            pltpu.VMEM_SHARED((NPAD, HH), jnp.float32),
            pltpu.VMEM_SHARED((NPAD, 16), jnp.float32),
        ],
        compiler_params=pltpu.CompilerParams(needs_layout_passes=False),
    )


def _sc_scatter(p_lo, p_hi, aux, dst16, zh, zs):
    del zh, zs
    dst = dst16.reshape(-1)
    ulo = jax.ops.segment_sum(p_lo, dst, num_segments=NPAD)
    uhi = jax.ops.segment_sum(p_hi, dst, num_segments=NPAD)
    sd = jax.ops.segment_sum(aux, dst, num_segments=NPAD)
    return ulo, uhi, sd


def _full(shape):
    return pl.BlockSpec(shape, lambda i: tuple(0 for _ in shape))


def _tc_init_body(h_ref, wd_ref, ws_ref, ta_ref, tb_ref):
    h = h_ref[...]
    ta_ref[...] = jnp.dot(h, wd_ref[...])
    tb_ref[...] = jnp.dot(h, ws_ref[...])


def _tc_init(h, wd, ws):
    nb = pl.cdiv(N, BLKN)
    return pl.pallas_call(
        _tc_init_body,
        grid=(nb,),
        in_specs=[
            pl.BlockSpec((BLKN, H), lambda i: (i, 0)),
            _full((H, H)),
            _full((H, H)),
        ],
        out_specs=[
            pl.BlockSpec((BLKN, H), lambda i: (i, 0)),
            pl.BlockSpec((BLKN, H), lambda i: (i, 0)),
        ],
        out_shape=[
            jax.ShapeDtypeStruct((N, H), jnp.float32),
            jax.ShapeDtypeStruct((N, H), jnp.float32),
        ],
    )(h, wd, ws)


def _tc_edge_body(td_ref, ts_ref, rd_ref, ea_ref, wdist_ref, wea_ref, be1_ref,
                  we2_ref, be2_ref, wa_ref, ba_ref, wc1_ref, bc1_ref,
                  wc2_ref, p_ref, q_ref, aux_ref):
    g = td_ref[...] + ts_ref[...]
    rd = rd_ref[...]
    rel = rd[:, 0:3]
    dist = rd[:, 3:4]
    m1 = g + dist * wdist_ref[...] + jnp.dot(ea_ref[...], wea_ref[...]) + be1_ref[...]
    m = _silu(jnp.dot(_silu(m1), we2_ref[...]) + be2_ref[...])
    alpha = jax.nn.sigmoid(jnp.dot(m, wa_ref[...]) + ba_ref[...])
    ex = jnp.exp(alpha)
    cw = jnp.dot(_silu(jnp.dot(m, wc1_ref[...]) + bc1_ref[...]), wc2_ref[...])
    p = ex * m
    p_ref[...] = p[:, :HH]
    q_ref[...] = p[:, HH:]
    dv = (ex * cw) * rel
    aux_ref[...] = jnp.concatenate(
        [ex, dv, jnp.zeros((ex.shape[0], 12), jnp.float32)], axis=1)


def _tc_edge(td, ts, rd, edge_attr, w):
    nb = E // BLKE
    return pl.pallas_call(
        _tc_edge_body,
        grid=(nb,),
        in_specs=[
            pl.BlockSpec((BLKE, H), lambda i: (i, 0)),
            pl.BlockSpec((BLKE, H), lambda i: (i, 0)),
            pl.BlockSpec((BLKE, 16), lambda i: (i, 0)),
            pl.BlockSpec((BLKE, 4), lambda i: (i, 0)),
            _full((1, H)),
            _full((4, H)),
            _full((1, H)),
            _full((H, H)),
            _full((1, H)),
            _full((H, 1)),
            _full((1, 1)),
            _full((H, H)),
            _full((1, H)),
            _full((H, 1)),
        ],
        out_specs=[
            pl.BlockSpec((BLKE, HH), lambda i: (i, 0)),
            pl.BlockSpec((BLKE, HH), lambda i: (i, 0)),
            pl.BlockSpec((BLKE, 16), lambda i: (i, 0)),
        ],
        out_shape=[
            jax.ShapeDtypeStruct((E, HH), jnp.float32),
            jax.ShapeDtypeStruct((E, HH), jnp.float32),
            jax.ShapeDtypeStruct((E, 16), jnp.float32),
        ],
    )(td, ts, rd, edge_attr, w["wdist"], w["wea"], w["be1"], w["we2"], w["be2"],
      w["wa"], w["ba"], w["wc1"], w["bc1"], w["wc2"])


def _finalize_common(ulo_ref, uhi_ref, sd_ref, h_ref, x16_ref, wn1h_ref,
                     wn1m_ref, bn1_ref, wn2_ref, bn2_ref):
    u = jnp.concatenate([ulo_ref[...], uhi_ref[...]], axis=1)
    sd = sd_ref[...]
    inv = 1.0 / (sd[:, 0:1] + 1e-16)
    mi = u * inv
    blk = u.shape[0]
    xn = x16_ref[...] + jnp.concatenate(
        [sd[:, 1:4] * inv, jnp.zeros((blk, 13), jnp.float32)], axis=1)
    h = h_ref[...]
    pre = jnp.dot(h, wn1h_ref[...]) + jnp.dot(mi, wn1m_ref[...]) + bn1_ref[...]
    hn = jnp.dot(_silu(pre), wn2_ref[...]) + bn2_ref[...]
    return hn, xn, h


def _tc_final_body(ulo_ref, uhi_ref, sd_ref, h_ref, x16_ref, wn1h_ref,
                   wn1m_ref, bn1_ref, wn2_ref, bn2_ref, wd_ref, ws_ref,
                   hn_ref, xn_ref, ta_ref, tb_ref, *, residual):
    hn, xn, h = _finalize_common(ulo_ref, uhi_ref, sd_ref, h_ref, x16_ref,
                                 wn1h_ref, wn1m_ref, bn1_ref, wn2_ref, bn2_ref)
    if residual:
        hn = hn + h
    hn_ref[...] = hn
    xn_ref[...] = xn
    ta_ref[...] = jnp.dot(hn, wd_ref[...])
    tb_ref[...] = jnp.dot(hn, ws_ref[...])


def _tc_final(ulo, uhi, sd, h, x16, w, wd_next, ws_next, residual):
    nb = pl.cdiv(N, BLKN)
    return pl.pallas_call(
        functools.partial(_tc_final_body, residual=residual),
        grid=(nb,),
        in_specs=[
            pl.BlockSpec((BLKN, HH), lambda i: (i, 0)),
            pl.BlockSpec((BLKN, HH), lambda i: (i, 0)),
            pl.BlockSpec((BLKN, 16), lambda i: (i, 0)),
            pl.BlockSpec((BLKN, H), lambda i: (i, 0)),
            pl.BlockSpec((BLKN, 16), lambda i: (i, 0)),
            _full((H, H)),
            _full((H, H)),
            _full((1, H)),
            _full((H, H)),
            _full((1, H)),
            _full((H, H)),
            _full((H, H)),
        ],
        out_specs=[
            pl.BlockSpec((BLKN, H), lambda i: (i, 0)),
            pl.BlockSpec((BLKN, 16), lambda i: (i, 0)),
            pl.BlockSpec((BLKN, H), lambda i: (i, 0)),
            pl.BlockSpec((BLKN, H), lambda i: (i, 0)),
        ],
        out_shape=[
            jax.ShapeDtypeStruct((N, H), jnp.float32),
            jax.ShapeDtypeStruct((N, 16), jnp.float32),
            jax.ShapeDtypeStruct((N, H), jnp.float32),
            jax.ShapeDtypeStruct((N, H), jnp.float32),
        ],
    )(ulo, uhi, sd, h, x16, w["wn1h"], w["wn1m"], w["bn1"], w["wn2"], w["bn2"],
      wd_next, ws_next)


def _tc_final_cls_body(ulo_ref, uhi_ref, sd_ref, h_ref, x16_ref, wn1h_ref,
                       wn1m_ref, bn1_ref, wn2_ref, bn2_ref, w1_ref, b1_ref,
                       w2_ref, b2_ref, logit_ref, xn_ref):
    hn, xn, _ = _finalize_common(ulo_ref, uhi_ref, sd_ref, h_ref, x16_ref,
                                 wn1h_ref, wn1m_ref, bn1_ref, wn2_ref, bn2_ref)
    t = _silu(jnp.dot(hn, w1_ref[...]) + b1_ref[...])
    logit_ref[...] = jnp.dot(t, w2_ref[...]) + b2_ref[...]
    xn_ref[...] = xn


def _tc_final_cls(ulo, uhi, sd, h, x16, w, cls):
    nb = pl.cdiv(N, BLKN)
    return pl.pallas_call(
        _tc_final_cls_body,
        grid=(nb,),
        in_specs=[
            pl.BlockSpec((BLKN, HH), lambda i: (i, 0)),
            pl.BlockSpec((BLKN, HH), lambda i: (i, 0)),
            pl.BlockSpec((BLKN, 16), lambda i: (i, 0)),
            pl.BlockSpec((BLKN, H), lambda i: (i, 0)),
            pl.BlockSpec((BLKN, 16), lambda i: (i, 0)),
            _full((H, H)),
            _full((H, H)),
            _full((1, H)),
            _full((H, H)),
            _full((1, H)),
            _full((H, H // 2)),
            _full((1, H // 2)),
            _full((H // 2, 1)),
            _full((1, 1)),
        ],
        out_specs=[
            pl.BlockSpec((BLKN, 1), lambda i: (i, 0)),
            pl.BlockSpec((BLKN, 16), lambda i: (i, 0)),
        ],
        out_shape=[
            jax.ShapeDtypeStruct((N, 1), jnp.float32),
            jax.ShapeDtypeStruct((N, 16), jnp.float32),
        ],
    )(ulo, uhi, sd, h, x16, w["wn1h"], w["wn1m"], w["bn1"], w["wn2"], w["bn2"],
      cls["W1"], cls["b1"].reshape(1, -1), cls["W2"], cls["b2"].reshape(1, 1))


def _layer_weights(p):
    we1 = p["We1"]
    return {
        "wd": we1[:H],
        "ws": we1[H:2 * H],
        "wdist": we1[2 * H:2 * H + 1],
        "wea": we1[2 * H + 1:],
        "be1": p["be1"].reshape(1, H),
        "we2": p["We2"], "be2": p["be2"].reshape(1, H),
        "wa": p["Wa"], "ba": p["ba"].reshape(1, 1),
        "wc1": p["Wc1"], "bc1": p["bc1"].reshape(1, H),
        "wc2": p["Wc2"],
        "wn1h": p["Wn1"][:H], "wn1m": p["Wn1"][H:],
        "bn1": p["bn1"].reshape(1, H),
        "wn2": p["Wn2"], "bn2": p["bn2"].reshape(1, H),
    }


def kernel(x, coords, edge_index, edge_attr, params):
    src = edge_index[0]
    dst = edge_index[1]
    dst3d = dst.reshape(NW, NCH, CH)
    src3d = src.reshape(NW, NCH, CH)
    dst16 = dst.reshape(NS, ECH, CH)
    zh = jnp.zeros((NPAD, HH), jnp.float32)
    zs = jnp.zeros((NPAD, 16), jnp.float32)
    x16 = jnp.pad(coords, ((0, 0), (0, 13)))
    lw = [_layer_weights(p) for p in params["layers"]]

    h = x
    ta, tb = _tc_init(h, lw[0]["wd"], lw[0]["ws"])
    logits = None
    for l in range(NUM_LAYERS):
        cflat = x16[:, :4].reshape(-1)
        td, ts, rd = _sc_gather(ta, tb, cflat, dst3d, src3d)
        p_lo, p_hi, aux = _tc_edge(td, ts, rd, edge_attr, lw[l])
        ulo, uhi, sd = _sc_scatter(p_lo, p_hi, aux, dst16, zh, zs)
        if l < NUM_LAYERS - 1:
            h, x16, ta, tb = _tc_final(
                ulo, uhi, sd, h, x16, lw[l], lw[l + 1]["wd"], lw[l + 1]["ws"],
                residual=(l != 0))
        else:
            logits, x16 = _tc_final_cls(ulo, uhi, sd, h, x16, lw[l],
                                        params["cls"])
    return logits, x16[:, :3]
